# baseline (speedup 1.0000x reference)
import jax
import jax.numpy as jnp
from jax.experimental import pallas as pl
from jax.experimental.pallas import tpu as pltpu

_I = False

B, S, DIM, K, D, L, H = 4, 1024, 13, 8192, 512, 2, 8
DH = D // H
N = B * S
TT = 512


def _mm(a, b, dims):
    return jax.lax.dot_general(a.astype(jnp.bfloat16), b.astype(jnp.bfloat16),
                               dims, preferred_element_type=jnp.float32)


def _l2n(t):
    return t / jnp.maximum(jnp.sqrt(jnp.sum(t * t, axis=-1, keepdims=True)), 1e-12)


def _ln(h, scale):
    mu = jnp.mean(h, axis=-1, keepdims=True)
    var = jnp.mean((h - mu) ** 2, axis=-1, keepdims=True)
    return (h - mu) / jnp.sqrt(var + 1e-5) * scale


def _prior_kernel(h0_ref, pos_ref, wqkv_ref, wo_ref, ln1_ref, ln2_ref,
                  w1_ref, w2_ref, lnf_ref, wout_ref, vn_ref):
    h = h0_ref[0] + pos_ref[...]
    QB = 256
    NQ = S // QB
    rio = jax.lax.broadcasted_iota(jnp.int32, (QB, QB), 0)
    cio = jax.lax.broadcasted_iota(jnp.int32, (QB, QB), 1)
    dmask = rio >= cio
    for l in range(L):
        a = _ln(h, ln1_ref[l:l + 1, :])
        qkv = _mm(a, wqkv_ref[l], (((1,), (0,)), ((), ())))
        rows = []
        for j in range(NQ):
            kend = (j + 1) * QB
            o_heads = []
            for hh in range(H):
                qh = qkv[j * QB:kend, hh * DH:(hh + 1) * DH]
                kh = qkv[:kend, D + hh * DH:D + (hh + 1) * DH]
                vh = qkv[:kend, 2 * D + hh * DH:2 * D + (hh + 1) * DH]
                att = _mm(qh, kh, (((1,), (1,)), ((), ())))
                att = att / jnp.sqrt(float(DH))
                if j == 0:
                    att = jnp.where(dmask, att, -1e9)
                else:
                    att = jnp.concatenate(
                        [att[:, :j * QB],
                         jnp.where(dmask, att[:, j * QB:], -1e9)], axis=1)
                att = jax.nn.softmax(att, axis=-1)
                o_heads.append(_mm(att, vh, (((1,), (0,)), ((), ()))))
            o_j = jnp.concatenate(o_heads, axis=1)
            rows.append(h[j * QB:kend, :] +
                        _mm(o_j, wo_ref[l], (((1,), (0,)), ((), ()))))
        h = jnp.concatenate(rows, axis=0)
        mid = _ln(h, ln2_ref[l:l + 1, :])
        hid = jax.nn.gelu(_mm(mid, w1_ref[l], (((1,), (0,)), ((), ()))))
        h = h + _mm(hid, w2_ref[l], (((1,), (0,)), ((), ())))
    v = _mm(_ln(h, lnf_ref[...]), wout_ref[...], (((1,), (0,)), ((), ())))
    vn_ref[0] = _l2n(v)


def _nll_kernel(vn_ref, qn_ref, cbn_ref, nll_ref):
    t = pl.program_id(0)
    vn = vn_ref[...]
    logits = _mm(vn, cbn_ref[...], (((1,), (1,)), ((), ())))
    m = jnp.max(logits, axis=-1, keepdims=True)
    lse = m + jnp.log(jnp.sum(jnp.exp(logits - m), axis=-1, keepdims=True))
    ltgt = jnp.sum(vn * qn_ref[...], axis=-1, keepdims=True)
    gi = t * TT + jax.lax.broadcasted_iota(jnp.int32, (TT, 1), 0)
    valid = (jnp.mod(gi, S) != (S - 1)).astype(jnp.float32)
    nll_ref[...] = jnp.zeros_like(nll_ref) + jnp.sum(valid * (lse - ltgt))


def kernel(x, codebook, W_in, pos_emb, Wqkv, Wo, ln1, ln2, W1, W2, lnf, W_out):
    commitment_weight = 0.25
    prior_loss_weight = 1.0
    xn = _l2n(x)
    cbn = _l2n(codebook)
    sim = jnp.einsum('bsd,cd->bsc', xn, cbn)
    indices = jnp.argmax(sim, axis=-1)
    quant = jnp.take(cbn, indices, axis=0)
    commit = jnp.mean((jax.lax.stop_gradient(quant) - xn) ** 2)
    vq_loss = commitment_weight * commit
    quantized_sg = xn + jax.lax.stop_gradient(quant - xn)
    Z_hat = jnp.take(cbn, indices, axis=0)
    prior_in = Z_hat[:, :-1, :] @ W_in
    uniq = jnp.unique(indices, size=K, fill_value=-1)
    usage = jnp.sum(uniq >= 0).astype(jnp.float32) / K * 100.0

    h0 = jnp.pad(prior_in, ((0, 0), (0, 1), (0, 0)))
    vn = pl.pallas_call(
        _prior_kernel,
        grid=(B,),
        in_specs=[
            pl.BlockSpec((1, S, D), lambda b: (b, 0, 0)),
            pl.BlockSpec((S, D), lambda b: (0, 0)),
            pl.BlockSpec((L, D, 3 * D), lambda b: (0, 0, 0)),
            pl.BlockSpec((L, D, D), lambda b: (0, 0, 0)),
            pl.BlockSpec((L, D), lambda b: (0, 0)),
            pl.BlockSpec((L, D), lambda b: (0, 0)),
            pl.BlockSpec((L, D, 4 * D), lambda b: (0, 0, 0)),
            pl.BlockSpec((L, 4 * D, D), lambda b: (0, 0, 0)),
            pl.BlockSpec((1, D), lambda b: (0, 0)),
            pl.BlockSpec((D, DIM), lambda b: (0, 0)),
        ],
        out_specs=pl.BlockSpec((1, S, DIM), lambda b: (b, 0, 0)),
        out_shape=jax.ShapeDtypeStruct((B, S, DIM), jnp.float32),
        compiler_params=pltpu.CompilerParams(
            dimension_semantics=("parallel",)),
        interpret=_I,
    )(h0, pos_emb, Wqkv, Wo, ln1, ln2, W1, W2, lnf.reshape(1, D), W_out)

    vnf = vn.reshape(N, DIM)
    qf = quant.reshape(N, DIM)
    qnext = jnp.concatenate([qf[1:], qf[:1]], axis=0)
    nt = N // TT
    nll_sum = pl.pallas_call(
        _nll_kernel,
        grid=(nt,),
        in_specs=[
            pl.BlockSpec((TT, DIM), lambda i: (i, 0)),
            pl.BlockSpec((TT, DIM), lambda i: (i, 0)),
            pl.BlockSpec((K, DIM), lambda i: (0, 0)),
        ],
        out_specs=pl.BlockSpec((1, 1, 1), lambda i: (i, 0, 0)),
        out_shape=jax.ShapeDtypeStruct((nt, 1, 1), jnp.float32),
        compiler_params=pltpu.CompilerParams(
            dimension_semantics=("parallel",)),
        interpret=_I,
    )(vnf, qnext, cbn)

    prior_loss = prior_loss_weight * (jnp.sum(nll_sum) / float(B * (S - 1)))
    return (quantized_sg, indices, vq_loss, prior_loss, commit, usage)


# speedup vs baseline: 1.1814x; 1.1814x over previous
import jax
import jax.numpy as jnp
from jax.experimental import pallas as pl
from jax.experimental.pallas import tpu as pltpu

_I = False

B, S, DIM, K, D, L, H = 4, 1024, 13, 8192, 512, 2, 8
DH = D // H
N = B * S
TT = 512


def _mm(a, b, dims):
    return jax.lax.dot_general(a.astype(jnp.bfloat16), b.astype(jnp.bfloat16),
                               dims, preferred_element_type=jnp.float32)


def _l2n(t):
    return t / jnp.maximum(jnp.sqrt(jnp.sum(t * t, axis=-1, keepdims=True)), 1e-12)


def _ln(h, scale):
    mu = jnp.mean(h, axis=-1, keepdims=True)
    var = jnp.mean((h - mu) ** 2, axis=-1, keepdims=True)
    return (h - mu) / jnp.sqrt(var + 1e-5) * scale


def _prior_kernel(h0_ref, pos_ref, wqkv_ref, wo_ref, ln1_ref, ln2_ref,
                  w1_ref, w2_ref, lnf_ref, wout_ref, vn_ref):
    h = h0_ref[0] + pos_ref[...]
    QB = 512
    NQ = S // QB
    rio = jax.lax.broadcasted_iota(jnp.int32, (QB, QB), 0)
    cio = jax.lax.broadcasted_iota(jnp.int32, (QB, QB), 1)
    dmask = rio >= cio
    for l in range(L):
        a = _ln(h, ln1_ref[l:l + 1, :])
        qkv = _mm(a, wqkv_ref[l], (((1,), (0,)), ((), ())))
        rows = []
        for j in range(NQ):
            kend = (j + 1) * QB
            o_heads = []
            for hh in range(H):
                qh = qkv[j * QB:kend, hh * DH:(hh + 1) * DH]
                kh = qkv[:kend, D + hh * DH:D + (hh + 1) * DH]
                vh = qkv[:kend, 2 * D + hh * DH:2 * D + (hh + 1) * DH]
                att = _mm(qh, kh, (((1,), (1,)), ((), ())))
                att = att / jnp.sqrt(float(DH))
                if j == 0:
                    att = jnp.where(dmask, att, -1e9)
                else:
                    att = jnp.concatenate(
                        [att[:, :j * QB],
                         jnp.where(dmask, att[:, j * QB:], -1e9)], axis=1)
                att = jax.nn.softmax(att, axis=-1)
                o_heads.append(_mm(att, vh, (((1,), (0,)), ((), ()))))
            o_j = jnp.concatenate(o_heads, axis=1)
            rows.append(h[j * QB:kend, :] +
                        _mm(o_j, wo_ref[l], (((1,), (0,)), ((), ()))))
        h = jnp.concatenate(rows, axis=0)
        mid = _ln(h, ln2_ref[l:l + 1, :])
        hid = jax.nn.gelu(_mm(mid, w1_ref[l], (((1,), (0,)), ((), ()))))
        h = h + _mm(hid, w2_ref[l], (((1,), (0,)), ((), ())))
    v = _mm(_ln(h, lnf_ref[...]), wout_ref[...], (((1,), (0,)), ((), ())))
    vn_ref[0] = _l2n(v)


def _nll_kernel(vn_ref, qn_ref, cbn_ref, nll_ref):
    t = pl.program_id(0)
    vn = vn_ref[...]
    logits = _mm(vn, cbn_ref[...], (((1,), (1,)), ((), ())))
    m = jnp.max(logits, axis=-1, keepdims=True)
    lse = m + jnp.log(jnp.sum(jnp.exp(logits - m), axis=-1, keepdims=True))
    ltgt = jnp.sum(vn * qn_ref[...], axis=-1, keepdims=True)
    gi = t * TT + jax.lax.broadcasted_iota(jnp.int32, (TT, 1), 0)
    valid = (jnp.mod(gi, S) != (S - 1)).astype(jnp.float32)
    nll_ref[...] = jnp.zeros_like(nll_ref) + jnp.sum(valid * (lse - ltgt))


def kernel(x, codebook, W_in, pos_emb, Wqkv, Wo, ln1, ln2, W1, W2, lnf, W_out):
    commitment_weight = 0.25
    prior_loss_weight = 1.0
    xn = _l2n(x)
    cbn = _l2n(codebook)
    sim = jnp.einsum('bsd,cd->bsc', xn, cbn)
    indices = jnp.argmax(sim, axis=-1)
    quant = jnp.take(cbn, indices, axis=0)
    commit = jnp.mean((jax.lax.stop_gradient(quant) - xn) ** 2)
    vq_loss = commitment_weight * commit
    quantized_sg = xn + jax.lax.stop_gradient(quant - xn)
    Z_hat = jnp.take(cbn, indices, axis=0)
    prior_in = Z_hat[:, :-1, :] @ W_in
    uniq = jnp.unique(indices, size=K, fill_value=-1)
    usage = jnp.sum(uniq >= 0).astype(jnp.float32) / K * 100.0

    h0 = jnp.pad(prior_in, ((0, 0), (0, 1), (0, 0)))
    vn = pl.pallas_call(
        _prior_kernel,
        grid=(B,),
        in_specs=[
            pl.BlockSpec((1, S, D), lambda b: (b, 0, 0)),
            pl.BlockSpec((S, D), lambda b: (0, 0)),
            pl.BlockSpec((L, D, 3 * D), lambda b: (0, 0, 0)),
            pl.BlockSpec((L, D, D), lambda b: (0, 0, 0)),
            pl.BlockSpec((L, D), lambda b: (0, 0)),
            pl.BlockSpec((L, D), lambda b: (0, 0)),
            pl.BlockSpec((L, D, 4 * D), lambda b: (0, 0, 0)),
            pl.BlockSpec((L, 4 * D, D), lambda b: (0, 0, 0)),
            pl.BlockSpec((1, D), lambda b: (0, 0)),
            pl.BlockSpec((D, DIM), lambda b: (0, 0)),
        ],
        out_specs=pl.BlockSpec((1, S, DIM), lambda b: (b, 0, 0)),
        out_shape=jax.ShapeDtypeStruct((B, S, DIM), jnp.float32),
        compiler_params=pltpu.CompilerParams(
            dimension_semantics=("parallel",)),
        interpret=_I,
    )(h0, pos_emb, Wqkv, Wo, ln1, ln2, W1, W2, lnf.reshape(1, D), W_out)

    vnf = vn.reshape(N, DIM)
    qf = quant.reshape(N, DIM)
    qnext = jnp.concatenate([qf[1:], qf[:1]], axis=0)
    nt = N // TT
    nll_sum = pl.pallas_call(
        _nll_kernel,
        grid=(nt,),
        in_specs=[
            pl.BlockSpec((TT, DIM), lambda i: (i, 0)),
            pl.BlockSpec((TT, DIM), lambda i: (i, 0)),
            pl.BlockSpec((K, DIM), lambda i: (0, 0)),
        ],
        out_specs=pl.BlockSpec((1, 1, 1), lambda i: (i, 0, 0)),
        out_shape=jax.ShapeDtypeStruct((nt, 1, 1), jnp.float32),
        compiler_params=pltpu.CompilerParams(
            dimension_semantics=("parallel",)),
        interpret=_I,
    )(vnf, qnext, cbn)

    prior_loss = prior_loss_weight * (jnp.sum(nll_sum) / float(B * (S - 1)))
    return (quantized_sg, indices, vq_loss, prior_loss, commit, usage)


# speedup vs baseline: 1.1824x; 1.0008x over previous
import jax
import jax.numpy as jnp
from jax.experimental import pallas as pl
from jax.experimental.pallas import tpu as pltpu

B, S, DIM, K, D, L, H = 4, 1024, 13, 8192, 512, 2, 8
DH = D // H
N = B * S
TT = 512


def _mm(a, b, dims):
    return jax.lax.dot_general(a.astype(jnp.bfloat16), b.astype(jnp.bfloat16),
                               dims, preferred_element_type=jnp.float32)


def _l2n(t):
    return t / jnp.maximum(jnp.sqrt(jnp.sum(t * t, axis=-1, keepdims=True)), 1e-12)


def _ln(h, scale):
    mu = jnp.mean(h, axis=-1, keepdims=True)
    var = jnp.mean((h - mu) ** 2, axis=-1, keepdims=True)
    return (h - mu) / jnp.sqrt(var + 1e-5) * scale


def _prior_kernel(h0_ref, pos_ref, wqkv_ref, wo_ref, ln1_ref, ln2_ref,
                  w1_ref, w2_ref, lnf_ref, wout_ref, vn_ref):
    h = h0_ref[0] + pos_ref[...]
    QB = 512
    NQ = S // QB
    rio = jax.lax.broadcasted_iota(jnp.int32, (QB, QB), 0)
    cio = jax.lax.broadcasted_iota(jnp.int32, (QB, QB), 1)
    dmask = rio >= cio
    for l in range(L):
        a = _ln(h, ln1_ref[l:l + 1, :])
        qkv = _mm(a, wqkv_ref[l], (((1,), (0,)), ((), ())))
        rows = []
        for j in range(NQ):
            kend = (j + 1) * QB
            o_heads = []
            for hh in range(H):
                qh = qkv[j * QB:kend, hh * DH:(hh + 1) * DH]
                kh = qkv[:kend, D + hh * DH:D + (hh + 1) * DH]
                vh = qkv[:kend, 2 * D + hh * DH:2 * D + (hh + 1) * DH]
                att = _mm(qh, kh, (((1,), (1,)), ((), ())))
                att = att / jnp.sqrt(float(DH))
                if j == 0:
                    att = jnp.where(dmask, att, -1e9)
                else:
                    att = jnp.concatenate(
                        [att[:, :j * QB],
                         jnp.where(dmask, att[:, j * QB:], -1e9)], axis=1)
                att = jax.nn.softmax(att, axis=-1)
                o_heads.append(_mm(att, vh, (((1,), (0,)), ((), ()))))
            o_j = jnp.concatenate(o_heads, axis=1)
            rows.append(h[j * QB:kend, :] +
                        _mm(o_j, wo_ref[l], (((1,), (0,)), ((), ()))))
        h = jnp.concatenate(rows, axis=0)
        mid = _ln(h, ln2_ref[l:l + 1, :])
        hid = jax.nn.gelu(_mm(mid, w1_ref[l], (((1,), (0,)), ((), ()))))
        h = h + _mm(hid, w2_ref[l], (((1,), (0,)), ((), ())))
    v = _mm(_ln(h, lnf_ref[...]), wout_ref[...], (((1,), (0,)), ((), ())))
    vn_ref[0] = _l2n(v)


def _nll_kernel(vn_ref, qn_ref, cbn_ref, nll_ref):
    t = pl.program_id(0)
    vn = vn_ref[...]
    logits = _mm(vn, cbn_ref[...], (((1,), (1,)), ((), ())))
    m = jnp.max(logits, axis=-1, keepdims=True)
    lse = m + jnp.log(jnp.sum(jnp.exp(logits - m), axis=-1, keepdims=True))
    ltgt = jnp.sum(vn * qn_ref[...], axis=-1, keepdims=True)
    gi = t * TT + jax.lax.broadcasted_iota(jnp.int32, (TT, 1), 0)
    valid = (jnp.mod(gi, S) != (S - 1)).astype(jnp.float32)
    nll_ref[...] = jnp.zeros_like(nll_ref) + jnp.sum(valid * (lse - ltgt))


def kernel(x, codebook, W_in, pos_emb, Wqkv, Wo, ln1, ln2, W1, W2, lnf, W_out):
    commitment_weight = 0.25
    prior_loss_weight = 1.0
    xn = _l2n(x)
    cbn = _l2n(codebook)
    sim = jnp.einsum('bsd,cd->bsc', xn, cbn)
    indices = jnp.argmax(sim, axis=-1)
    quant = jnp.take(cbn, indices, axis=0)
    commit = jnp.mean((jax.lax.stop_gradient(quant) - xn) ** 2)
    vq_loss = commitment_weight * commit
    quantized_sg = xn + jax.lax.stop_gradient(quant - xn)
    Z_hat = jnp.take(cbn, indices, axis=0)
    prior_in = Z_hat[:, :-1, :] @ W_in
    uniq = jnp.unique(indices, size=K, fill_value=-1)
    usage = jnp.sum(uniq >= 0).astype(jnp.float32) / K * 100.0

    h0 = jnp.pad(prior_in, ((0, 0), (0, 1), (0, 0)))
    vn = pl.pallas_call(
        _prior_kernel,
        grid=(B,),
        in_specs=[
            pl.BlockSpec((1, S, D), lambda b: (b, 0, 0)),
            pl.BlockSpec((S, D), lambda b: (0, 0)),
            pl.BlockSpec((L, D, 3 * D), lambda b: (0, 0, 0)),
            pl.BlockSpec((L, D, D), lambda b: (0, 0, 0)),
            pl.BlockSpec((L, D), lambda b: (0, 0)),
            pl.BlockSpec((L, D), lambda b: (0, 0)),
            pl.BlockSpec((L, D, 4 * D), lambda b: (0, 0, 0)),
            pl.BlockSpec((L, 4 * D, D), lambda b: (0, 0, 0)),
            pl.BlockSpec((1, D), lambda b: (0, 0)),
            pl.BlockSpec((D, DIM), lambda b: (0, 0)),
        ],
        out_specs=pl.BlockSpec((1, S, DIM), lambda b: (b, 0, 0)),
        out_shape=jax.ShapeDtypeStruct((B, S, DIM), jnp.float32),
        compiler_params=pltpu.CompilerParams(
            dimension_semantics=("parallel",)),
    )(h0, pos_emb, Wqkv, Wo, ln1, ln2, W1, W2, lnf.reshape(1, D), W_out)

    vnf = vn.reshape(N, DIM)
    qf = quant.reshape(N, DIM)
    qnext = jnp.concatenate([qf[1:], qf[:1]], axis=0)
    nt = N // TT
    nll_sum = pl.pallas_call(
        _nll_kernel,
        grid=(nt,),
        in_specs=[
            pl.BlockSpec((TT, DIM), lambda i: (i, 0)),
            pl.BlockSpec((TT, DIM), lambda i: (i, 0)),
            pl.BlockSpec((K, DIM), lambda i: (0, 0)),
        ],
        out_specs=pl.BlockSpec((1, 1, 1), lambda i: (i, 0, 0)),
        out_shape=jax.ShapeDtypeStruct((nt, 1, 1), jnp.float32),
        compiler_params=pltpu.CompilerParams(
            dimension_semantics=("parallel",)),
    )(vnf, qnext, cbn)

    prior_loss = prior_loss_weight * (jnp.sum(nll_sum) / float(B * (S - 1)))
    return (quantized_sg, indices, vq_loss, prior_loss, commit, usage)
